# baseline (speedup 1.0000x reference)
import functools

import jax
import jax.numpy as jnp
import numpy as np
from jax import lax
from jax.experimental import pallas as pl
from jax.experimental.pallas import tpu as pltpu
from jax.experimental.pallas import tpu_sc as plsc

_FIELD_DIMS = [38462] * 26
_NUM_HEADS = 2

_NC = 2
_NS = 16
_NW = _NC * _NS


def _build(batch, num_fields, total_rows):
  b_per_w = batch // _NW
  mesh = plsc.VectorSubcoreMesh(core_axis_name="c", subcore_axis_name="s")

  @functools.partial(
      pl.kernel,
      out_type=jax.ShapeDtypeStruct((batch, 4, num_fields, 8), jnp.float32),
      mesh=mesh,
      scratch_types=[
          pltpu.VMEM((b_per_w, num_fields), jnp.int32),
          pltpu.VMEM((b_per_w, num_fields, 32), jnp.float32),
          pltpu.SemaphoreType.DMA,
      ],
      compiler_params=pltpu.CompilerParams(use_tc_tiling_on_sc=False),
  )
  def gather_kernel(idx_hbm, table_hbm, out_hbm, idx_v, rows_v, sem):
    wid = lax.axis_index("s") * _NC + lax.axis_index("c")
    base = wid * b_per_w

    pltpu.sync_copy(idx_hbm.at[pl.ds(base, b_per_w), :], idx_v)

    @pl.loop(0, b_per_w, unroll=8)
    def _fire(b):
      pltpu.async_copy(table_hbm.at[idx_v.at[b]], rows_v.at[b], sem)

    @pl.loop(0, b_per_w, unroll=8)
    def _drain(b):
      pltpu.make_async_copy(table_hbm.at[idx_v.at[b]], rows_v.at[b], sem).wait()

    for q in range(4):
      pltpu.sync_copy(
          rows_v.at[:, :, pl.ds(q * 8, 8)],
          out_hbm.at[pl.ds(base, b_per_w), q, :, :],
      )

  return gather_kernel


def kernel(x, table):
  batch, num_fields = x.shape
  total_rows, embed_dim = table.shape
  offsets = jnp.asarray(
      np.concatenate(([0], np.cumsum(_FIELD_DIMS)[:-1])), dtype=x.dtype
  )
  idx = x + offsets[None, :]
  out4 = _build(batch, num_fields, total_rows)(idx, table)
  return out4.reshape(batch, _NUM_HEADS, (4 // _NUM_HEADS) * num_fields * 8)


# speedup vs baseline: 1.3466x; 1.3466x over previous
import functools

import jax
import jax.numpy as jnp
import numpy as np
from jax import lax
from jax.experimental import pallas as pl
from jax.experimental.pallas import tpu as pltpu
from jax.experimental.pallas import tpu_sc as plsc

_FIELD_DIMS = [38462] * 26
_NUM_HEADS = 2

_NC = 2
_NS = 16
_NW = _NC * _NS
_CHUNK = 32


def _build(batch, num_fields, total_rows):
  b_per_w = batch // _NW
  n_chunks = b_per_w // _CHUNK
  n_tiles = 4 * num_fields
  mesh = plsc.VectorSubcoreMesh(core_axis_name="c", subcore_axis_name="s")

  @functools.partial(
      pl.kernel,
      out_type=jax.ShapeDtypeStruct((n_tiles, _NW, 8, 128), jnp.float32),
      mesh=mesh,
      scratch_types=[
          pltpu.VMEM((b_per_w, num_fields), jnp.int32),
          pltpu.VMEM((_CHUNK, num_fields, 32), jnp.float32),
          pltpu.VMEM((_CHUNK, num_fields, 32), jnp.float32),
          pltpu.VMEM((n_tiles, 8, _CHUNK), jnp.float32),
          pltpu.SemaphoreType.DMA,
          pltpu.SemaphoreType.DMA,
      ],
      compiler_params=pltpu.CompilerParams(
          use_tc_tiling_on_sc=False, needs_layout_passes=False),
  )
  def gather_kernel(idx_hbm, table_hbm, out_hbm, idx_v, rows0, rows1, out_v,
                    sem0, sem1):
    wid = lax.axis_index("s") * _NC + lax.axis_index("c")
    base = wid * b_per_w

    pltpu.sync_copy(idx_hbm.at[pl.ds(base, b_per_w), :], idx_v)

    rows_bufs = (rows0, rows1)
    sems = (sem0, sem1)

    def fire(s, buf, sem):
      @pl.loop(0, _CHUNK, unroll=8)
      def _fire(b):
        pltpu.async_copy(
            table_hbm.at[idx_v.at[s * _CHUNK + b]], buf.at[b], sem)

    def drain(s, buf, sem):
      @pl.loop(0, _CHUNK, unroll=8)
      def _drain(b):
        pltpu.make_async_copy(
            table_hbm.at[idx_v.at[s * _CHUNK + b]], buf.at[b], sem).wait()

    def transpose_and_store(s, buf):
      @pl.loop(0, n_tiles * 8)
      def _tp(i):
        tk = i // 8
        j = i % 8
        q = tk // num_fields
        f = tk - q * num_fields
        c = 8 * q + j
        lane = jax.lax.iota(jnp.int32, 16)
        f_vec = jnp.full((16,), f, dtype=jnp.int32)
        c_vec = jnp.full((16,), c, dtype=jnp.int32)
        for t in range(_CHUNK // 16):
          vals = plsc.load_gather(buf, [lane + 16 * t, f_vec, c_vec])
          out_v[tk, j, pl.ds(16 * t, 16)] = vals

      pltpu.sync_copy(
          out_v, out_hbm.at[:, wid, :, pl.ds(s * _CHUNK, _CHUNK)])

    fire(0, rows_bufs[0], sems[0])
    for s in range(n_chunks):
      if s + 1 < n_chunks:
        fire(s + 1, rows_bufs[(s + 1) % 2], sems[(s + 1) % 2])
      drain(s, rows_bufs[s % 2], sems[s % 2])
      transpose_and_store(s, rows_bufs[s % 2])

  return gather_kernel


def kernel(x, table):
  batch, num_fields = x.shape
  total_rows, embed_dim = table.shape
  offsets = jnp.asarray(
      np.concatenate(([0], np.cumsum(_FIELD_DIMS)[:-1])), dtype=x.dtype
  )
  idx = x + offsets[None, :]
  o4 = _build(batch, num_fields, total_rows)(idx, table)
  out = o4.transpose(1, 3, 0, 2).reshape(batch, 2 * num_fields * 16)
  return out.reshape(batch, _NUM_HEADS, num_fields * 16)
